# baseline (speedup 1.0000x reference)
import functools

import jax
import jax.numpy as jnp
from jax import lax
from jax.experimental import pallas as pl
from jax.experimental.pallas import tpu as pltpu
from jax.experimental.pallas import tpu_sc as plsc

N_NODE = 1_000_000
DIM = 64
PDIM = 128
BATCH = 16384
L = 16
NC, NS = 2, 16
NW = NC * NS
BPW = BATCH // NW
CH = 128
NCHUNK = BPW // CH
BIAS_PAD = 1000448

_mesh = plsc.VectorSubcoreMesh(core_axis_name="c", subcore_axis_name="s")


@functools.partial(
    pl.kernel,
    mesh=_mesh,
    out_type=[
        jax.ShapeDtypeStruct((BATCH, PDIM), jnp.float32),
        jax.ShapeDtypeStruct((BATCH, PDIM), jnp.float32),
        jax.ShapeDtypeStruct((BATCH,), jnp.float32),
    ],
    scratch_types=[
        pltpu.VMEM((CH,), jnp.int32),
        pltpu.VMEM((CH,), jnp.int32),
        pltpu.VMEM((CH,), jnp.int32),
        pltpu.VMEM((CH,), jnp.int32),
        pltpu.VMEM((CH, PDIM), jnp.float32),
        pltpu.VMEM((CH, PDIM), jnp.float32),
        pltpu.VMEM((CH,), jnp.float32),
        pltpu.VMEM((BPW,), jnp.float32),
        pltpu.SemaphoreType.DMA,
    ],
)
def _gan_kernel(ext, biasp, ida, idb, out_a, out_b, out_p,
                idx_a, idx_b, pid_a, pid_b, rows_a, rows_b, bias_v,
                prob_v, sem):
    wid = lax.axis_index("s") * NC + lax.axis_index("c")
    base = wid * BPW

    lane = lax.iota(jnp.int32, L)
    perms = [(lane ^ sh).reshape(L, 1) for sh in (8, 4, 2, 1)]
    dnums = lax.GatherDimensionNumbers(
        offset_dims=(), collapsed_slice_dims=(0,), start_index_map=(0,))

    def hsum(v):
        for p in perms:
            v = v + lax.gather(v, p, dnums, (1,),
                               mode=lax.GatherScatterMode.PROMISE_IN_BOUNDS)
        return v

    for h in range(NCHUNK):
        cbase = base + h * CH
        pltpu.sync_copy(ida.at[pl.ds(cbase, CH)], idx_a)
        pltpu.sync_copy(idb.at[pl.ds(cbase, CH)], idx_b)
        for v in range(CH // L):
            sl = pl.ds(v * L, L)
            pid_a[sl] = lax.shift_right_logical(idx_a[sl], 1)
            pid_b[sl] = lax.shift_right_logical(idx_b[sl], 1)

        c1 = pltpu.async_copy(ext.at[pid_a], rows_a, sem)
        c2 = pltpu.async_copy(ext.at[pid_b], rows_b, sem)
        c3 = pltpu.async_copy(biasp.at[idx_b], bias_v, sem)
        c1.wait()
        c2.wait()
        c3.wait()

        def group_body(g, carry):
            sl = pl.ds(g * L, L)
            offs_a = (idx_a[sl] & 1) * DIM
            offs_b = (idx_b[sl] & 1) * DIM
            s = jnp.zeros((L,), jnp.float32)
            for k in range(L):
                r = g * L + k
                off_a = offs_a[k]
                off_b = offs_b[k]
                acc = None
                for c in range(DIM // L):
                    va = rows_a[r, pl.ds(off_a + c * L, L)]
                    vb = rows_b[r, pl.ds(off_b + c * L, L)]
                    rows_a[r, pl.ds(c * L, L)] = va
                    rows_b[r, pl.ds(c * L, L)] = vb
                    acc = va * vb if acc is None else acc + va * vb
                s = jnp.where(lane == k, hsum(acc), s)
            s = s + bias_v[sl]
            p = 1.0 / (1.0 + jnp.exp(-s))
            p = jnp.minimum(jnp.maximum(p, 1e-5), 1.0)
            prob_v[pl.ds(h * CH + g * L, L)] = p
            return carry

        lax.fori_loop(0, CH // L, group_body, 0)

        pltpu.sync_copy(rows_a, out_a.at[pl.ds(cbase, CH)])
        pltpu.sync_copy(rows_b, out_b.at[pl.ds(cbase, CH)])

    pltpu.sync_copy(prob_v, out_p.at[pl.ds(base, BPW)])


def kernel(embedding_matrix, bias_vector, node_id, node_neighbor_id):
    ext = embedding_matrix.reshape(N_NODE // 2, 2 * DIM)
    biasp = jnp.pad(bias_vector, (0, BIAS_PAD - N_NODE))
    rows_a, rows_b, prob = _gan_kernel(ext, biasp, node_id, node_neighbor_id)
    return (rows_a[:, :DIM], rows_b[:, :DIM], prob)


# speedup vs baseline: 1.6665x; 1.6665x over previous
import functools

import jax
import jax.numpy as jnp
from jax import lax
from jax.experimental import pallas as pl
from jax.experimental.pallas import tpu as pltpu
from jax.experimental.pallas import tpu_sc as plsc

N_NODE = 1_000_000
DIM = 64
BATCH = 16384
L = 16
NC, NS = 2, 16
NW = NC * NS
BPW = BATCH // NW
CH = 128
NCHUNK = BPW // CH
BIAS_PAD = 1000448

_mesh = plsc.VectorSubcoreMesh(core_axis_name="c", subcore_axis_name="s")


@functools.partial(
    pl.kernel,
    mesh=_mesh,
    out_type=[
        jax.ShapeDtypeStruct((BATCH, DIM), jnp.float32),
        jax.ShapeDtypeStruct((BATCH, DIM), jnp.float32),
        jax.ShapeDtypeStruct((BATCH,), jnp.float32),
    ],
    scratch_types=[
        pltpu.VMEM((NCHUNK, CH), jnp.int32),
        pltpu.VMEM((NCHUNK, CH), jnp.int32),
        pltpu.VMEM((CH, DIM), jnp.float32),
        pltpu.VMEM((CH, DIM), jnp.float32),
        pltpu.VMEM((CH, DIM), jnp.float32),
        pltpu.VMEM((CH, DIM), jnp.float32),
        pltpu.VMEM((BPW,), jnp.float32),
        pltpu.VMEM((BPW,), jnp.float32),
        pltpu.SemaphoreType.DMA,
        pltpu.SemaphoreType.DMA,
        pltpu.SemaphoreType.DMA,
    ],
)
def _gan_kernel(emb, biasp, ida, idb, out_a, out_b, out_p,
                idx_va, idx_v, a0, a1, b0, b1, bias_v, prob_v,
                sem0, sem1, semb):
    wid = lax.axis_index("s") * NC + lax.axis_index("c")
    base = wid * BPW
    abuf = (a0, a1)
    bbuf = (b0, b1)
    sems = (sem0, sem1)

    for j in range(NCHUNK):
        pltpu.sync_copy(ida.at[pl.ds(base + j * CH, CH)], idx_va.at[j])
        pltpu.sync_copy(idb.at[pl.ds(base + j * CH, CH)], idx_v.at[j])

    bias_copies = [
        pltpu.async_copy(biasp.at[idx_v.at[j]],
                         bias_v.at[pl.ds(j * CH, CH)], semb)
        for j in range(NCHUNK)
    ]

    def fire_chunk(c, buf):
        ra, rb, sem = abuf[buf], bbuf[buf], sems[buf]

        def issue(g, carry):
            idv_a = idx_va[c, pl.ds(g * L, L)]
            idv_b = idx_v[c, pl.ds(g * L, L)]
            for k in range(L):
                pltpu.async_copy(emb.at[pl.ds(idv_a[k], 1)],
                                 ra.at[pl.ds(g * L + k, 1)], sem)
                pltpu.async_copy(emb.at[pl.ds(idv_b[k], 1)],
                                 rb.at[pl.ds(g * L + k, 1)], sem)
            return carry

        lax.fori_loop(0, CH // L, issue, 0)

    def drain_chunk(buf):
        ra, rb, sem = abuf[buf], bbuf[buf], sems[buf]
        for r in range(CH):
            pltpu.make_async_copy(emb.at[pl.ds(0, 1)], ra.at[pl.ds(r, 1)],
                                  sem).wait()
            pltpu.make_async_copy(emb.at[pl.ds(0, 1)], rb.at[pl.ds(r, 1)],
                                  sem).wait()

    lane = lax.iota(jnp.int32, L)
    perms = [(lane ^ sh).reshape(L, 1) for sh in (8, 4, 2, 1)]
    dnums = lax.GatherDimensionNumbers(
        offset_dims=(), collapsed_slice_dims=(0,), start_index_map=(0,))

    def hsum(v):
        for p in perms:
            v = v + lax.gather(v, p, dnums, (1,),
                               mode=lax.GatherScatterMode.PROMISE_IN_BOUNDS)
        return v

    def compute_chunk(c, buf):
        ra, rb = abuf[buf], bbuf[buf]

        def group_body(g, carry):
            s = jnp.zeros((L,), jnp.float32)
            for k in range(L):
                r = g * L + k
                acc = ra[r, pl.ds(0, L)] * rb[r, pl.ds(0, L)]
                for cc in range(1, DIM // L):
                    acc = acc + ra[r, pl.ds(cc * L, L)] * rb[r, pl.ds(cc * L, L)]
                s = jnp.where(lane == k, hsum(acc), s)
            s = s + bias_v[pl.ds(c * CH + g * L, L)]
            p = 1.0 / (1.0 + jnp.exp(-s))
            p = jnp.minimum(jnp.maximum(p, 1e-5), 1.0)
            prob_v[pl.ds(c * CH + g * L, L)] = p
            return carry

        lax.fori_loop(0, CH // L, group_body, 0)
        pltpu.sync_copy(ra, out_a.at[pl.ds(base + c * CH, CH)])
        pltpu.sync_copy(rb, out_b.at[pl.ds(base + c * CH, CH)])

    fire_chunk(0, 0)
    for c in range(NCHUNK):
        buf = c & 1
        drain_chunk(buf)
        if c + 1 < NCHUNK:
            fire_chunk(c + 1, (c + 1) & 1)
        if c == 0:
            for bc in bias_copies:
                bc.wait()
        compute_chunk(c, buf)

    pltpu.sync_copy(prob_v, out_p.at[pl.ds(base, BPW)])


def kernel(embedding_matrix, bias_vector, node_id, node_neighbor_id):
    biasp = jnp.pad(bias_vector, (0, BIAS_PAD - N_NODE))
    rows_a, rows_b, prob = _gan_kernel(
        embedding_matrix, biasp, node_id, node_neighbor_id)
    return (rows_a, rows_b, prob)


# speedup vs baseline: 2.3546x; 1.4129x over previous
import functools

import jax
import jax.numpy as jnp
from jax import lax
from jax.experimental import pallas as pl
from jax.experimental.pallas import tpu as pltpu
from jax.experimental.pallas import tpu_sc as plsc

N_NODE = 1_000_000
DIM = 64
BATCH = 16384
L = 16
NC, NS = 2, 16
NW = NC * NS
BPW = BATCH // NW
CH = 128
NCHUNK = BPW // CH
BIAS_PAD = 1000448

_mesh = plsc.VectorSubcoreMesh(core_axis_name="c", subcore_axis_name="s")


@functools.partial(
    pl.kernel,
    mesh=_mesh,
    out_type=[
        jax.ShapeDtypeStruct((BATCH, DIM), jnp.float32),
        jax.ShapeDtypeStruct((BATCH, DIM), jnp.float32),
        jax.ShapeDtypeStruct((BATCH,), jnp.float32),
    ],
    scratch_types=[
        pltpu.VMEM((NCHUNK, CH), jnp.int32),
        pltpu.VMEM((NCHUNK, CH), jnp.int32),
        pltpu.VMEM((CH, DIM), jnp.float32),
        pltpu.VMEM((CH, DIM), jnp.float32),
        pltpu.VMEM((CH, DIM), jnp.float32),
        pltpu.VMEM((CH, DIM), jnp.float32),
        pltpu.VMEM((BPW,), jnp.float32),
        pltpu.VMEM((BPW,), jnp.float32),
        pltpu.SemaphoreType.DMA,
        pltpu.SemaphoreType.DMA,
        pltpu.SemaphoreType.DMA,
    ],
)
def _gan_kernel(emb, biasp, ida, idb, out_a, out_b, out_p,
                idx_va, idx_v, a0, a1, b0, b1, bias_v, prob_v,
                sem0, sem1, semb):
    wid = lax.axis_index("s") * NC + lax.axis_index("c")
    base = wid * BPW
    abuf = (a0, a1)
    bbuf = (b0, b1)
    sems = (sem0, sem1)

    for j in range(NCHUNK):
        pltpu.sync_copy(ida.at[pl.ds(base + j * CH, CH)], idx_va.at[j])
        pltpu.sync_copy(idb.at[pl.ds(base + j * CH, CH)], idx_v.at[j])

    bias_copies = [
        pltpu.async_copy(biasp.at[idx_v.at[j]],
                         bias_v.at[pl.ds(j * CH, CH)], semb)
        for j in range(NCHUNK)
    ]

    def fire_chunk(c, buf):
        ra, rb, sem = abuf[buf], bbuf[buf], sems[buf]

        def issue(g, carry):
            idv_a = idx_va[c, pl.ds(g * L, L)]
            idv_b = idx_v[c, pl.ds(g * L, L)]
            for k in range(L):
                pltpu.async_copy(emb.at[pl.ds(idv_a[k], 1)],
                                 ra.at[pl.ds(g * L + k, 1)], sem)
                pltpu.async_copy(emb.at[pl.ds(idv_b[k], 1)],
                                 rb.at[pl.ds(g * L + k, 1)], sem)
            return carry

        lax.fori_loop(0, CH // L, issue, 0)

    def drain_chunk(buf):
        ra, rb, sem = abuf[buf], bbuf[buf], sems[buf]
        for r in range(CH):
            pltpu.make_async_copy(emb.at[pl.ds(0, 1)], ra.at[pl.ds(r, 1)],
                                  sem).wait()
            pltpu.make_async_copy(emb.at[pl.ds(0, 1)], rb.at[pl.ds(r, 1)],
                                  sem).wait()

    lane = lax.iota(jnp.int32, L)
    perms = [(lane ^ sh).reshape(L, 1) for sh in (8, 4, 2, 1)]
    dnums = lax.GatherDimensionNumbers(
        offset_dims=(), collapsed_slice_dims=(0,), start_index_map=(0,))

    def hsum(v):
        for p in perms:
            v = v + lax.gather(v, p, dnums, (1,),
                               mode=lax.GatherScatterMode.PROMISE_IN_BOUNDS)
        return v

    def compute_chunk(c, buf):
        ra, rb = abuf[buf], bbuf[buf]

        def group_body(g, carry):
            s = jnp.zeros((L,), jnp.float32)
            for k in range(L):
                r = g * L + k
                acc = ra[r, pl.ds(0, L)] * rb[r, pl.ds(0, L)]
                for cc in range(1, DIM // L):
                    acc = acc + ra[r, pl.ds(cc * L, L)] * rb[r, pl.ds(cc * L, L)]
                s = jnp.where(lane == k, hsum(acc), s)
            s = s + bias_v[pl.ds(c * CH + g * L, L)]
            p = 1.0 / (1.0 + jnp.exp(-s))
            p = jnp.minimum(jnp.maximum(p, 1e-5), 1.0)
            prob_v[pl.ds(c * CH + g * L, L)] = p
            return carry

        lax.fori_loop(0, CH // L, group_body, 0)
        pltpu.sync_copy(ra, out_a.at[pl.ds(base + c * CH, CH)])
        pltpu.sync_copy(rb, out_b.at[pl.ds(base + c * CH, CH)])

    fire_chunk(0, 0)
    for c in range(NCHUNK):
        buf = c & 1
        drain_chunk(buf)
        if c + 1 < NCHUNK:
            fire_chunk(c + 1, (c + 1) & 1)
        if c == 0:
            for bc in bias_copies:
                bc.wait()
        compute_chunk(c, buf)

    pltpu.sync_copy(prob_v, out_p.at[pl.ds(base, BPW)])


def kernel(embedding_matrix, bias_vector, node_id, node_neighbor_id):
    emb = lax.optimization_barrier(embedding_matrix.T).T
    biasp = jnp.pad(bias_vector, (0, BIAS_PAD - N_NODE))
    rows_a, rows_b, prob = _gan_kernel(
        emb, biasp, node_id, node_neighbor_id)
    return (rows_a, rows_b, prob)
